# Initial kernel scaffold; baseline (speedup 1.0000x reference)
import jax
import jax.numpy as jnp
from jax.experimental import pallas as pl


def kernel(y_true, y_pred, confusion_matrix):
    raise NotImplementedError("write your pallas kernel here")


# speedup vs baseline: 1.9607x; 1.9607x over previous
import functools

import jax
import jax.numpy as jnp
from jax.experimental import pallas as pl
from jax.experimental.pallas import tpu as pltpu

_EPS = float(jnp.finfo(jnp.float32).eps)


def _argmax_onehot(x, iota, c):
    m = jnp.max(x, axis=1, keepdims=True)
    idx = jnp.min(jnp.where(x == m, iota, c), axis=1, keepdims=True)
    return (iota == idx).astype(jnp.bfloat16)


def _body(yt_ref, yp_ref, cm_ref, out_ref, acc_ref, *, nsteps, c):
    i = pl.program_id(0)

    @pl.when(i == 0)
    def _init():
        acc_ref[...] = jnp.zeros_like(acc_ref)

    b = yt_ref.shape[0]
    iota = jax.lax.broadcasted_iota(jnp.int32, (b, c), 1)
    t_oh = _argmax_onehot(yt_ref[...], iota, c)
    p_oh = _argmax_onehot(yp_ref[...], iota, c)
    acc_ref[...] += jax.lax.dot_general(
        t_oh, p_oh, (((0,), (0,)), ((), ())),
        preferred_element_type=jnp.float32)

    @pl.when(i == nsteps - 1)
    def _finish():
        cm = acc_ref[...] + cm_ref[...]
        r_iota = jax.lax.broadcasted_iota(jnp.int32, (c, c), 0)
        c_iota = jax.lax.broadcasted_iota(jnp.int32, (c, c), 1)
        tp = jnp.sum(jnp.where(r_iota == c_iota, cm, 0.0), axis=1)
        rowsum = jnp.sum(cm, axis=1)
        out_ref[0] = jnp.sum(tp / (rowsum + _EPS)) * (1.0 / c)


def _pick_block(n):
    for b in (10000, 5000, 4000, 2500, 2000, 1000, 500, 250, 200, 100, 50,
              40, 25, 20, 10, 8, 5, 4, 2, 1):
        if n % b == 0:
            return b
    return n


@jax.jit
def kernel(y_true, y_pred, confusion_matrix):
    n, c = y_true.shape
    b = _pick_block(n)
    nsteps = n // b
    out = pl.pallas_call(
        functools.partial(_body, nsteps=nsteps, c=c),
        grid=(nsteps,),
        in_specs=[
            pl.BlockSpec((b, c), lambda i: (i, 0)),
            pl.BlockSpec((b, c), lambda i: (i, 0)),
            pl.BlockSpec((c, c), lambda i: (0, 0)),
        ],
        out_specs=pl.BlockSpec(memory_space=pltpu.SMEM),
        out_shape=jax.ShapeDtypeStruct((1,), jnp.float32),
        scratch_shapes=[pltpu.VMEM((c, c), jnp.float32)],
    )(y_true, y_pred, confusion_matrix)
    return out[0]


# speedup vs baseline: 4.5828x; 2.3373x over previous
import functools

import jax
import jax.numpy as jnp
from jax.experimental import pallas as pl
from jax.experimental.pallas import tpu as pltpu

_EPS = float(jnp.finfo(jnp.float32).eps)


def _argmax_onehot(x, tri, c):
    m = jnp.max(x, axis=1, keepdims=True)
    hits = (x == m).astype(jnp.bfloat16)
    runcount = jax.lax.dot_general(
        hits, tri, (((1,), (0,)), ((), ())),
        preferred_element_type=jnp.float32)
    return jnp.where(runcount == 1.0, hits, jnp.bfloat16(0.0))


def _body(yt_ref, yp_ref, cm_ref, out_ref, acc_ref, *, nsteps, c):
    i = pl.program_id(0)

    @pl.when(i == 0)
    def _init():
        acc_ref[...] = jnp.zeros_like(acc_ref)

    r_iota = jax.lax.broadcasted_iota(jnp.int32, (c, c), 0)
    c_iota = jax.lax.broadcasted_iota(jnp.int32, (c, c), 1)
    tri = (r_iota <= c_iota).astype(jnp.bfloat16)
    t_oh = _argmax_onehot(yt_ref[...], tri, c)
    p_oh = _argmax_onehot(yp_ref[...], tri, c)
    acc_ref[...] += jax.lax.dot_general(
        t_oh, p_oh, (((0,), (0,)), ((), ())),
        preferred_element_type=jnp.float32)

    @pl.when(i == nsteps - 1)
    def _finish():
        cm = acc_ref[...] + cm_ref[...]
        r_iota = jax.lax.broadcasted_iota(jnp.int32, (c, c), 0)
        c_iota = jax.lax.broadcasted_iota(jnp.int32, (c, c), 1)
        tp = jnp.sum(jnp.where(r_iota == c_iota, cm, 0.0), axis=1)
        rowsum = jnp.sum(cm, axis=1)
        out_ref[0] = jnp.sum(tp / (rowsum + _EPS)) * (1.0 / c)


def _pick_block(n):
    for b in (10000, 5000, 4000, 2500, 2000, 1000, 500, 250, 200, 100, 50,
              40, 25, 20, 10, 8, 5, 4, 2, 1):
        if n % b == 0:
            return b
    return n


@jax.jit
def kernel(y_true, y_pred, confusion_matrix):
    n, c = y_true.shape
    b = _pick_block(n)
    nsteps = n // b
    out = pl.pallas_call(
        functools.partial(_body, nsteps=nsteps, c=c),
        grid=(nsteps,),
        in_specs=[
            pl.BlockSpec((b, c), lambda i: (i, 0)),
            pl.BlockSpec((b, c), lambda i: (i, 0)),
            pl.BlockSpec((c, c), lambda i: (0, 0)),
        ],
        out_specs=pl.BlockSpec(memory_space=pltpu.SMEM),
        out_shape=jax.ShapeDtypeStruct((1,), jnp.float32),
        scratch_shapes=[pltpu.VMEM((c, c), jnp.float32)],
    )(y_true, y_pred, confusion_matrix)
    return out[0]


# speedup vs baseline: 4.8968x; 1.0685x over previous
import functools

import jax
import jax.numpy as jnp
from jax.experimental import pallas as pl
from jax.experimental.pallas import tpu as pltpu

_EPS = float(jnp.finfo(jnp.float32).eps)


def _argmax_onehot(x, tri, c):
    m = jnp.max(x, axis=1, keepdims=True)
    hits = (x == m).astype(jnp.bfloat16)
    runcount = jax.lax.dot_general(
        hits, tri, (((1,), (0,)), ((), ())),
        preferred_element_type=jnp.float32)
    return jnp.where(runcount == 1.0, hits, jnp.bfloat16(0.0))


def _body(yt_ref, yp_ref, cm_ref, out_ref, acc_ref, *, nsteps, c):
    i = pl.program_id(0)

    @pl.when(i == 0)
    def _init():
        acc_ref[...] = jnp.zeros_like(acc_ref)

    r_iota = jax.lax.broadcasted_iota(jnp.int32, (c, c), 0)
    c_iota = jax.lax.broadcasted_iota(jnp.int32, (c, c), 1)
    tri = (r_iota <= c_iota).astype(jnp.bfloat16)
    t_oh = _argmax_onehot(yt_ref[...], tri, c)
    p_oh = _argmax_onehot(yp_ref[...], tri, c)
    acc_ref[...] += jax.lax.dot_general(
        t_oh, p_oh, (((0,), (0,)), ((), ())),
        preferred_element_type=jnp.float32)

    @pl.when(i == nsteps - 1)
    def _finish():
        cm = acc_ref[...] + cm_ref[...]
        r_iota = jax.lax.broadcasted_iota(jnp.int32, (c, c), 0)
        c_iota = jax.lax.broadcasted_iota(jnp.int32, (c, c), 1)
        tp = jnp.sum(jnp.where(r_iota == c_iota, cm, 0.0), axis=1)
        rowsum = jnp.sum(cm, axis=1)
        out_ref[0] = jnp.sum(tp / (rowsum + _EPS)) * (1.0 / c)


def _pick_block(n):
    for b in (20000, 10000, 5000, 4000, 2500, 2000, 1000, 500, 250, 200, 100, 50,
              40, 25, 20, 10, 8, 5, 4, 2, 1):
        if n % b == 0:
            return b
    return n


@jax.jit
def kernel(y_true, y_pred, confusion_matrix):
    n, c = y_true.shape
    b = _pick_block(n)
    nsteps = n // b
    out = pl.pallas_call(
        functools.partial(_body, nsteps=nsteps, c=c),
        grid=(nsteps,),
        in_specs=[
            pl.BlockSpec((b, c), lambda i: (i, 0)),
            pl.BlockSpec((b, c), lambda i: (i, 0)),
            pl.BlockSpec((c, c), lambda i: (0, 0)),
        ],
        out_specs=pl.BlockSpec(memory_space=pltpu.SMEM),
        out_shape=jax.ShapeDtypeStruct((1,), jnp.float32),
        scratch_shapes=[pltpu.VMEM((c, c), jnp.float32)],
    )(y_true, y_pred, confusion_matrix)
    return out[0]
